# Initial kernel scaffold; baseline (speedup 1.0000x reference)
import jax
import jax.numpy as jnp
from jax.experimental import pallas as pl


def kernel(x, edges_tensor, molecule_ids, embed_table, W_0_0, W_0_1, W_1_0, W_1_1, fc_w, fc_b):
    raise NotImplementedError("write your pallas kernel here")


# speedup vs baseline: 12.0691x; 12.0691x over previous
import jax
import jax.numpy as jnp
from jax import lax
from jax.experimental import pallas as pl
from jax.experimental.pallas import tpu as pltpu
from jax.experimental.pallas import tpu_sc as plsc

N = 100000
E = 1600000
NMOL = 1000
OUT = 32

NC = 2
NS = 16
NW = NC * NS

CH = 128
EB = 8
E_BLOCKS = 49
EPW = E_BLOCKS * EB * CH
E_PAD = NW * EPW
N_PAD = 102400
RPW = N_PAD // NW
RPS = N_PAD // NS
MOLP = 1024
MW = 25
MROWS = 32

_MESH = dict(core_axis_name="c", subcore_axis_name="s", num_cores=NC,
             num_subcores=NS)


def _worker_ids():
    c = lax.axis_index("c")
    s = lax.axis_index("s")
    return c, s, c * NS + s


def _hist_body(dst_hbm, mol_hbm, deg_out, cnt_out, deg_acc, cnt_acc, idx_v,
               ones_v, zrows):
    c, s, wid = _worker_ids()

    def fill_z(i, carry):
        zrows[i, :] = jnp.zeros((16,), jnp.float32)
        return carry

    lax.fori_loop(0, 256, fill_z, None)

    def fill_o(i, carry):
        ones_v[i, :] = jnp.ones((16,), jnp.float32)
        return carry

    lax.fori_loop(0, CH, fill_o, None)

    base = s * RPS

    def zero_acc(k, carry):
        pltpu.sync_copy(zrows, deg_acc.at[pl.ds(base + k * 256, 256)])
        return carry

    lax.fori_loop(0, RPS // 256, zero_acc, None)
    pltpu.sync_copy(zrows.at[pl.ds(0, MOLP // NS)],
                    cnt_acc.at[pl.ds(s * (MOLP // NS), MOLP // NS)])
    plsc.subcore_barrier()

    ebase = wid * (EPW // CH)

    def deg_block(b, carry):
        pltpu.sync_copy(dst_hbm.at[pl.ds(ebase + b * EB, EB)], idx_v)
        for j in range(EB):
            pltpu.sync_copy(ones_v, deg_acc.at[idx_v.at[j]], add=True)
        return carry

    lax.fori_loop(0, E_BLOCKS, deg_block, None)

    @pl.when(wid < MW)
    def _cnt_loop():
        mbase = wid * MROWS

        def cnt_block(b, carry):
            pltpu.sync_copy(mol_hbm.at[pl.ds(mbase + b * EB, EB)], idx_v)
            for j in range(EB):
                pltpu.sync_copy(ones_v, cnt_acc.at[idx_v.at[j]], add=True)
            return carry

        lax.fori_loop(0, MROWS // EB, cnt_block, None)

    plsc.subcore_barrier()

    def wr_deg(k, carry):
        pltpu.sync_copy(deg_acc.at[pl.ds(base + k * 256, 256)],
                        deg_out.at[c, pl.ds(base + k * 256, 256)])
        return carry

    lax.fori_loop(0, RPS // 256, wr_deg, None)
    pltpu.sync_copy(cnt_acc.at[pl.ds(s * (MOLP // NS), MOLP // NS)],
                    cnt_out.at[c, pl.ds(s * (MOLP // NS), MOLP // NS)])


def _hist_call(dst2d, mol2d):
    k = pl.kernel(
        _hist_body,
        out_type=(
            jax.ShapeDtypeStruct((NC, N_PAD, 16), jnp.float32),
            jax.ShapeDtypeStruct((NC, MOLP, 16), jnp.float32),
        ),
        mesh=plsc.VectorSubcoreMesh(**_MESH),
        compiler_params=pltpu.CompilerParams(use_tc_tiling_on_sc=False),
        scratch_types=[
            pltpu.VMEM_SHARED((N_PAD, 16), jnp.float32),
            pltpu.VMEM_SHARED((MOLP, 16), jnp.float32),
            pltpu.VMEM((EB, CH), jnp.int32),
            pltpu.VMEM((CH, 16), jnp.float32),
            pltpu.VMEM((256, 16), jnp.float32),
        ],
    )
    return k(dst2d, mol2d)


def _edge_body(src_hbm, dst_hbm, feat_hbm, out_hbm, acc, sidx, didx, rows,
               zrows, sem):
    c, s, wid = _worker_ids()

    def fill_z(i, carry):
        zrows[i, :] = jnp.zeros((16,), jnp.float32)
        return carry

    lax.fori_loop(0, 256, fill_z, None)

    base = s * RPS

    def zero_acc(k, carry):
        pltpu.sync_copy(zrows, acc.at[pl.ds(base + k * 256, 256)])
        return carry

    lax.fori_loop(0, RPS // 256, zero_acc, None)
    plsc.subcore_barrier()

    ebase = wid * (EPW // CH)

    def block(b, carry):
        pltpu.sync_copy(src_hbm.at[pl.ds(ebase + b * EB, EB)], sidx)
        pltpu.sync_copy(dst_hbm.at[pl.ds(ebase + b * EB, EB)], didx)
        descs = [
            pltpu.async_copy(feat_hbm.at[sidx.at[j]],
                             rows.at[pl.ds(j * CH, CH)], sem)
            for j in range(EB)
        ]
        for d in descs:
            d.wait()
        for j in range(EB):
            pltpu.sync_copy(rows.at[pl.ds(j * CH, CH)], acc.at[didx.at[j]],
                            add=True)
        return carry

    lax.fori_loop(0, E_BLOCKS, block, None)
    plsc.subcore_barrier()

    for k in range(6):
        pltpu.sync_copy(acc.at[pl.ds(base + k * 1024, 1024)],
                        out_hbm.at[c, pl.ds(base + k * 1024, 1024)])
    pltpu.sync_copy(acc.at[pl.ds(base + 6144, 256)],
                    out_hbm.at[c, pl.ds(base + 6144, 256)])


def _edge_call(src2d, dst2d, feat):
    k = pl.kernel(
        _edge_body,
        out_type=jax.ShapeDtypeStruct((NC, N_PAD, 16), jnp.float32),
        mesh=plsc.VectorSubcoreMesh(**_MESH),
        compiler_params=pltpu.CompilerParams(use_tc_tiling_on_sc=False),
        scratch_types=[
            pltpu.VMEM_SHARED((N_PAD, 16), jnp.float32),
            pltpu.VMEM((EB, CH), jnp.int32),
            pltpu.VMEM((EB, CH), jnp.int32),
            pltpu.VMEM((EB * CH, 16), jnp.float32),
            pltpu.VMEM((256, 16), jnp.float32),
            pltpu.SemaphoreType.DMA,
        ],
    )
    return k(src2d, dst2d, feat)


def _pool_body(y_hbm, mol_hbm, out_hbm, acc, midx, ybuf, zbuf):
    c, s, wid = _worker_ids()

    def fill_z(i, carry):
        zbuf[i, pl.ds(0, 16)] = jnp.zeros((16,), jnp.float32)
        zbuf[i, pl.ds(16, 16)] = jnp.zeros((16,), jnp.float32)
        return carry

    lax.fori_loop(0, MOLP // NS, fill_z, None)
    pltpu.sync_copy(zbuf, acc.at[pl.ds(s * (MOLP // NS), MOLP // NS)])
    plsc.subcore_barrier()

    @pl.when(wid < MW)
    def _pool_loop():
        pltpu.sync_copy(mol_hbm.at[pl.ds(wid * MROWS, MROWS)], midx)
        rbase = wid * MROWS * CH

        def chunk(b, carry):
            pltpu.sync_copy(y_hbm.at[pl.ds(rbase + b * CH, CH)], ybuf)
            pltpu.sync_copy(ybuf, acc.at[midx.at[b]], add=True)
            return carry

        lax.fori_loop(0, MROWS, chunk, None)

    plsc.subcore_barrier()
    pltpu.sync_copy(acc.at[pl.ds(s * (MOLP // NS), MOLP // NS)],
                    out_hbm.at[c, pl.ds(s * (MOLP // NS), MOLP // NS)])


def _pool_call(y, mol2d):
    k = pl.kernel(
        _pool_body,
        out_type=jax.ShapeDtypeStruct((NC, MOLP, OUT), jnp.float32),
        mesh=plsc.VectorSubcoreMesh(**_MESH),
        compiler_params=pltpu.CompilerParams(use_tc_tiling_on_sc=False),
        scratch_types=[
            pltpu.VMEM_SHARED((MOLP, OUT), jnp.float32),
            pltpu.VMEM((MROWS, CH), jnp.int32),
            pltpu.VMEM((CH, OUT), jnp.float32),
            pltpu.VMEM((MOLP // NS, OUT), jnp.float32),
        ],
    )
    return k(y, mol2d)


BB = 2048


def _embed_body(x_ref, emb_ref, out_ref):
    x = x_ref[...]
    oh = (x[:, None] == lax.broadcasted_iota(jnp.int32, (BB, 32), 1))
    out_ref[...] = jnp.dot(oh.astype(jnp.float32),
                           jnp.maximum(emb_ref[...], 0.0),
                           preferred_element_type=jnp.float32,
                           precision=lax.Precision.HIGHEST)


def _embed_call(x_p, embed_table):
    return pl.pallas_call(
        _embed_body,
        grid=(N_PAD // BB,),
        in_specs=[
            pl.BlockSpec((BB,), lambda i: (i,)),
            pl.BlockSpec((32, 16), lambda i: (0, 0)),
        ],
        out_specs=pl.BlockSpec((BB, 16), lambda i: (i, 0)),
        out_shape=jax.ShapeDtypeStruct((N_PAD, 16), jnp.float32),
    )(x_p, embed_table)


def _mlp_body(feat_ref, aggA_ref, aggB_ref, degA_ref, degB_ref, w0_ref,
              w1_ref, out_ref):
    r = 1.0 / jnp.maximum(degA_ref[...][:, 0] + degB_ref[...][:, 0], 1.0)
    m = jnp.concatenate([feat_ref[...], aggA_ref[...] + aggB_ref[...]],
                        axis=1) * r[:, None]
    h = jnp.maximum(jnp.dot(m, w0_ref[...],
                            preferred_element_type=jnp.float32), 0.0)
    out_ref[...] = jnp.maximum(jnp.dot(h, w1_ref[...],
                                       preferred_element_type=jnp.float32),
                               0.0)


def _mlp_call(feat, aggA, aggB, degA, degB, w0, w1):
    return pl.pallas_call(
        _mlp_body,
        grid=(N_PAD // BB,),
        in_specs=[
            pl.BlockSpec((BB, 16), lambda i: (i, 0)),
            pl.BlockSpec((BB, 16), lambda i: (i, 0)),
            pl.BlockSpec((BB, 16), lambda i: (i, 0)),
            pl.BlockSpec((BB, 16), lambda i: (i, 0)),
            pl.BlockSpec((BB, 16), lambda i: (i, 0)),
            pl.BlockSpec((32, 32), lambda i: (0, 0)),
            pl.BlockSpec((32, 16), lambda i: (0, 0)),
        ],
        out_specs=pl.BlockSpec((BB, 16), lambda i: (i, 0)),
        out_shape=jax.ShapeDtypeStruct((N_PAD, 16), jnp.float32),
    )(feat, aggA, aggB, degA, degB, w0, w1)


def _final_body(feat0_ref, msg0_ref, aggA_ref, aggB_ref, degA_ref, degB_ref,
                w0_ref, w1_ref, fc_ref, out_ref):
    r = 1.0 / jnp.maximum(degA_ref[...][:, 0] + degB_ref[...][:, 0], 1.0)
    m = jnp.concatenate([msg0_ref[...], aggA_ref[...] + aggB_ref[...]],
                        axis=1) * r[:, None]
    h = jnp.maximum(jnp.dot(m, w0_ref[...],
                            preferred_element_type=jnp.float32), 0.0)
    msg1 = jnp.maximum(jnp.dot(h, w1_ref[...],
                               preferred_element_type=jnp.float32), 0.0)
    fc = fc_ref[...]
    y = (jnp.dot(feat0_ref[...], fc[0:16], preferred_element_type=jnp.float32)
         + jnp.dot(msg0_ref[...], fc[16:32],
                   preferred_element_type=jnp.float32)
         + jnp.dot(msg1, fc[32:48], preferred_element_type=jnp.float32))
    out_ref[...] = y


def _final_call(feat0, msg0, aggA, aggB, degA, degB, w0, w1, fc_w):
    return pl.pallas_call(
        _final_body,
        grid=(N_PAD // BB,),
        in_specs=[
            pl.BlockSpec((BB, 16), lambda i: (i, 0)),
            pl.BlockSpec((BB, 16), lambda i: (i, 0)),
            pl.BlockSpec((BB, 16), lambda i: (i, 0)),
            pl.BlockSpec((BB, 16), lambda i: (i, 0)),
            pl.BlockSpec((BB, 16), lambda i: (i, 0)),
            pl.BlockSpec((BB, 16), lambda i: (i, 0)),
            pl.BlockSpec((32, 32), lambda i: (0, 0)),
            pl.BlockSpec((32, 16), lambda i: (0, 0)),
            pl.BlockSpec((48, 32), lambda i: (0, 0)),
        ],
        out_specs=pl.BlockSpec((BB, OUT), lambda i: (i, 0)),
        out_shape=jax.ShapeDtypeStruct((N_PAD, OUT), jnp.float32),
    )(feat0, msg0, aggA, aggB, degA, degB, w0, w1, fc_w)


def _finish_body(pA_ref, pB_ref, cA_ref, cB_ref, fcb_ref, out_ref):
    pool = pA_ref[...] + pB_ref[...]
    cnt = jnp.maximum(cA_ref[...][:, 0] + cB_ref[...][:, 0], 1.0)
    out_ref[...] = pool[:NMOL] / cnt[:NMOL, None] + fcb_ref[...][None, :]


def _finish_call(pA, pB, cA, cB, fc_b):
    return pl.pallas_call(
        _finish_body,
        grid=(1,),
        in_specs=[
            pl.BlockSpec((MOLP, OUT), lambda i: (0, 0)),
            pl.BlockSpec((MOLP, OUT), lambda i: (0, 0)),
            pl.BlockSpec((MOLP, 16), lambda i: (0, 0)),
            pl.BlockSpec((MOLP, 16), lambda i: (0, 0)),
            pl.BlockSpec((OUT,), lambda i: (0,)),
        ],
        out_specs=pl.BlockSpec((NMOL, OUT), lambda i: (0, 0)),
        out_shape=jax.ShapeDtypeStruct((NMOL, OUT), jnp.float32),
    )(pA, pB, cA, cB, fc_b)


def kernel(x, edges_tensor, molecule_ids, embed_table, W_0_0, W_0_1, W_1_0,
           W_1_1, fc_w, fc_b):
    x_p = jnp.pad(x.astype(jnp.int32), (0, N_PAD - N))
    mol2d = jnp.pad(molecule_ids.astype(jnp.int32), (0, N_PAD - N),
                    constant_values=NMOL).reshape(N_PAD // CH, CH)
    src2d = jnp.pad(edges_tensor[0].astype(jnp.int32), (0, E_PAD - E),
                    constant_values=N).reshape(E_PAD // CH, CH)
    dst2d = jnp.pad(edges_tensor[1].astype(jnp.int32), (0, E_PAD - E),
                    constant_values=N).reshape(E_PAD // CH, CH)

    deg2, cnt2 = _hist_call(dst2d, mol2d)
    feat0 = _embed_call(x_p, embed_table)
    agg0 = _edge_call(src2d, dst2d, feat0)
    msg0 = _mlp_call(feat0, agg0[0], agg0[1], deg2[0], deg2[1], W_0_0, W_0_1)
    agg1 = _edge_call(src2d, dst2d, msg0)
    y = _final_call(feat0, msg0, agg1[0], agg1[1], deg2[0], deg2[1], W_1_0,
                    W_1_1, fc_w)
    pool2 = _pool_call(y, mol2d)
    return _finish_call(pool2[0], pool2[1], cnt2[0], cnt2[1], fc_b)


# speedup vs baseline: 18.4632x; 1.5298x over previous
import jax
import jax.numpy as jnp
from jax import lax
from jax.experimental import pallas as pl
from jax.experimental.pallas import tpu as pltpu
from jax.experimental.pallas import tpu_sc as plsc

N = 100000
E = 1600000
NMOL = 1000
OUT = 32

NC = 2
NS = 16
NW = NC * NS

CH = 128
EB = 8
E_BLOCKS = 49
EPW = E_BLOCKS * EB * CH
E_PAD = NW * EPW
N_PAD = 102400
RPW = N_PAD // NW
RPS = N_PAD // NS
MOLP = 1024
MW = 25
MROWS = 32

_MESH = dict(core_axis_name="c", subcore_axis_name="s", num_cores=NC,
             num_subcores=NS)


def _worker_ids():
    c = lax.axis_index("c")
    s = lax.axis_index("s")
    return c, s, c * NS + s


def _hist_body(dst_hbm, mol_hbm, x_hbm, emb_hbm, deg_out, cnt_out, feat_out,
               deg_acc, cnt_acc, idx_v, ones_v, zrows, xrows, sem):
    c, s, wid = _worker_ids()

    def fill_z(i, carry):
        zrows[i, :] = jnp.zeros((16,), jnp.float32)
        return carry

    lax.fori_loop(0, 256, fill_z, None)

    def fill_o(i, carry):
        ones_v[i, :] = jnp.ones((16,), jnp.float32)
        return carry

    lax.fori_loop(0, CH, fill_o, None)

    base = s * RPS

    def zero_acc(k, carry):
        pltpu.sync_copy(zrows, deg_acc.at[pl.ds(base + k * 256, 256)])
        return carry

    lax.fori_loop(0, RPS // 256, zero_acc, None)
    pltpu.sync_copy(zrows.at[pl.ds(0, MOLP // NS)],
                    cnt_acc.at[pl.ds(s * (MOLP // NS), MOLP // NS)])
    plsc.subcore_barrier()

    @pl.when(wid < MW)
    def _embed_loop():
        def emb_block(b, carry):
            pltpu.sync_copy(x_hbm.at[pl.ds(wid * MROWS + b * EB, EB)], idx_v)
            descs = [
                pltpu.async_copy(emb_hbm.at[idx_v.at[j]],
                                 xrows.at[pl.ds(j * CH, CH)], sem)
                for j in range(EB)
            ]
            for d in descs:
                d.wait()

            def do_relu(i, carry2):
                xrows[i, :] = jnp.maximum(xrows[i, :], 0.0)
                return carry2

            lax.fori_loop(0, EB * CH, do_relu, None)
            pltpu.sync_copy(
                xrows, feat_out.at[pl.ds(wid * MROWS * CH + b * EB * CH,
                                         EB * CH)])
            return carry

        lax.fori_loop(0, MROWS // EB, emb_block, None)

    ebase = wid * (EPW // CH)

    def deg_block(b, carry):
        pltpu.sync_copy(dst_hbm.at[pl.ds(ebase + b * EB, EB)], idx_v)
        for j in range(EB):
            pltpu.sync_copy(ones_v, deg_acc.at[idx_v.at[j]], add=True)
        return carry

    lax.fori_loop(0, E_BLOCKS, deg_block, None)

    @pl.when(wid < MW)
    def _cnt_loop():
        mbase = wid * MROWS

        def cnt_block(b, carry):
            pltpu.sync_copy(mol_hbm.at[pl.ds(mbase + b * EB, EB)], idx_v)
            for j in range(EB):
                pltpu.sync_copy(ones_v, cnt_acc.at[idx_v.at[j]], add=True)
            return carry

        lax.fori_loop(0, MROWS // EB, cnt_block, None)

    plsc.subcore_barrier()

    def wr_deg(k, carry):
        pltpu.sync_copy(deg_acc.at[pl.ds(base + k * 256, 256)],
                        deg_out.at[c, pl.ds(base + k * 256, 256)])
        return carry

    lax.fori_loop(0, RPS // 256, wr_deg, None)
    pltpu.sync_copy(cnt_acc.at[pl.ds(s * (MOLP // NS), MOLP // NS)],
                    cnt_out.at[c, pl.ds(s * (MOLP // NS), MOLP // NS)])


def _hist_call(dst2d, mol2d, x2d, emb):
    k = pl.kernel(
        _hist_body,
        out_type=(
            jax.ShapeDtypeStruct((NC, N_PAD, 16), jnp.float32),
            jax.ShapeDtypeStruct((NC, MOLP, 16), jnp.float32),
            jax.ShapeDtypeStruct((N_PAD, 16), jnp.float32),
        ),
        mesh=plsc.VectorSubcoreMesh(**_MESH),
        compiler_params=pltpu.CompilerParams(use_tc_tiling_on_sc=False),
        scratch_types=[
            pltpu.VMEM_SHARED((N_PAD, 16), jnp.float32),
            pltpu.VMEM_SHARED((MOLP, 16), jnp.float32),
            pltpu.VMEM((EB, CH), jnp.int32),
            pltpu.VMEM((CH, 16), jnp.float32),
            pltpu.VMEM((256, 16), jnp.float32),
            pltpu.VMEM((EB * CH, 16), jnp.float32),
            pltpu.SemaphoreType.DMA,
        ],
    )
    return k(dst2d, mol2d, x2d, emb)


def _edge_body(src_hbm, dst_hbm, feat_hbm, out_hbm, acc, sidx, didx, rows,
               zrows, sem):
    c, s, wid = _worker_ids()

    def fill_z(i, carry):
        zrows[i, :] = jnp.zeros((16,), jnp.float32)
        return carry

    lax.fori_loop(0, 256, fill_z, None)

    base = s * RPS

    def zero_acc(k, carry):
        pltpu.sync_copy(zrows, acc.at[pl.ds(base + k * 256, 256)])
        return carry

    lax.fori_loop(0, RPS // 256, zero_acc, None)
    plsc.subcore_barrier()

    ebase = wid * (EPW // CH)

    def block(b, carry):
        pltpu.sync_copy(src_hbm.at[pl.ds(ebase + b * EB, EB)], sidx)
        pltpu.sync_copy(dst_hbm.at[pl.ds(ebase + b * EB, EB)], didx)
        descs = [
            pltpu.async_copy(feat_hbm.at[sidx.at[j]],
                             rows.at[pl.ds(j * CH, CH)], sem)
            for j in range(EB)
        ]
        for d in descs:
            d.wait()
        for j in range(EB):
            pltpu.sync_copy(rows.at[pl.ds(j * CH, CH)], acc.at[didx.at[j]],
                            add=True)
        return carry

    lax.fori_loop(0, E_BLOCKS, block, None)
    plsc.subcore_barrier()

    for k in range(6):
        pltpu.sync_copy(acc.at[pl.ds(base + k * 1024, 1024)],
                        out_hbm.at[c, pl.ds(base + k * 1024, 1024)])
    pltpu.sync_copy(acc.at[pl.ds(base + 6144, 256)],
                    out_hbm.at[c, pl.ds(base + 6144, 256)])


def _edge_call(src2d, dst2d, feat):
    k = pl.kernel(
        _edge_body,
        out_type=jax.ShapeDtypeStruct((NC, N_PAD, 16), jnp.float32),
        mesh=plsc.VectorSubcoreMesh(**_MESH),
        compiler_params=pltpu.CompilerParams(use_tc_tiling_on_sc=False),
        scratch_types=[
            pltpu.VMEM_SHARED((N_PAD, 16), jnp.float32),
            pltpu.VMEM((EB, CH), jnp.int32),
            pltpu.VMEM((EB, CH), jnp.int32),
            pltpu.VMEM((EB * CH, 16), jnp.float32),
            pltpu.VMEM((256, 16), jnp.float32),
            pltpu.SemaphoreType.DMA,
        ],
    )
    return k(src2d, dst2d, feat)


def _pool_body(y_hbm, mol_hbm, out_hbm, acc, midx, ybuf, zbuf):
    c, s, wid = _worker_ids()

    def fill_z(i, carry):
        zbuf[i, pl.ds(0, 16)] = jnp.zeros((16,), jnp.float32)
        zbuf[i, pl.ds(16, 16)] = jnp.zeros((16,), jnp.float32)
        return carry

    lax.fori_loop(0, MOLP // NS, fill_z, None)
    pltpu.sync_copy(zbuf, acc.at[pl.ds(s * (MOLP // NS), MOLP // NS)])
    plsc.subcore_barrier()

    @pl.when(wid < MW)
    def _pool_loop():
        pltpu.sync_copy(mol_hbm.at[pl.ds(wid * MROWS, MROWS)], midx)
        rbase = wid * MROWS * CH

        def chunk(b, carry):
            pltpu.sync_copy(y_hbm.at[pl.ds(rbase + b * CH, CH)], ybuf)
            pltpu.sync_copy(ybuf, acc.at[midx.at[b]], add=True)
            return carry

        lax.fori_loop(0, MROWS, chunk, None)

    plsc.subcore_barrier()
    pltpu.sync_copy(acc.at[pl.ds(s * (MOLP // NS), MOLP // NS)],
                    out_hbm.at[c, pl.ds(s * (MOLP // NS), MOLP // NS)])


def _pool_call(y, mol2d):
    k = pl.kernel(
        _pool_body,
        out_type=jax.ShapeDtypeStruct((NC, MOLP, OUT), jnp.float32),
        mesh=plsc.VectorSubcoreMesh(**_MESH),
        compiler_params=pltpu.CompilerParams(use_tc_tiling_on_sc=False),
        scratch_types=[
            pltpu.VMEM_SHARED((MOLP, OUT), jnp.float32),
            pltpu.VMEM((MROWS, CH), jnp.int32),
            pltpu.VMEM((CH, OUT), jnp.float32),
            pltpu.VMEM((MOLP // NS, OUT), jnp.float32),
        ],
    )
    return k(y, mol2d)


NPK = N_PAD // 8
BR = 512


def _mlp_body(xf_ref, xa2_ref_a, xa2_ref_b, dg_ref_a, dg_ref_b, k0t_ref,
              k0b_ref, k1_ref, out_ref):
    r = 1.0 / jnp.maximum(dg_ref_a[...][0] + dg_ref_b[...][0], 1.0)
    xf = xf_ref[...] * r
    xa = (xa2_ref_a[...][0] + xa2_ref_b[...][0]) * r
    h = jnp.maximum(
        jnp.dot(xf, k0t_ref[...], preferred_element_type=jnp.float32)
        + jnp.dot(xa, k0b_ref[...], preferred_element_type=jnp.float32), 0.0)
    out_ref[...] = jnp.maximum(
        jnp.dot(h, k1_ref[...], preferred_element_type=jnp.float32), 0.0)


def _mlp_call(featp, aggp, degp, k0t, k0b, k1):
    return pl.pallas_call(
        _mlp_body,
        grid=(NPK // BR,),
        in_specs=[
            pl.BlockSpec((BR, 128), lambda i: (i, 0)),
            pl.BlockSpec((1, BR, 128), lambda i: (0, i, 0)),
            pl.BlockSpec((1, BR, 128), lambda i: (1, i, 0)),
            pl.BlockSpec((1, BR, 128), lambda i: (0, i, 0)),
            pl.BlockSpec((1, BR, 128), lambda i: (1, i, 0)),
            pl.BlockSpec((128, 256), lambda i: (0, 0)),
            pl.BlockSpec((128, 256), lambda i: (0, 0)),
            pl.BlockSpec((256, 128), lambda i: (0, 0)),
        ],
        out_specs=pl.BlockSpec((BR, 128), lambda i: (i, 0)),
        out_shape=jax.ShapeDtypeStruct((NPK, 128), jnp.float32),
    )(featp, aggp, aggp, degp, degp, k0t, k0b, k1)


def _final_body(xf0_ref, xm0_ref, xa2_ref_a, xa2_ref_b, dg_ref_a, dg_ref_b,
                k1t_ref, k1b_ref, k2_ref, kfa_ref, kfb_ref, kfc_ref, out_ref):
    r = 1.0 / jnp.maximum(dg_ref_a[...][0] + dg_ref_b[...][0], 1.0)
    xm0 = xm0_ref[...]
    xm = xm0 * r
    xa = (xa2_ref_a[...][0] + xa2_ref_b[...][0]) * r
    h = jnp.maximum(
        jnp.dot(xm, k1t_ref[...], preferred_element_type=jnp.float32)
        + jnp.dot(xa, k1b_ref[...], preferred_element_type=jnp.float32), 0.0)
    m1 = jnp.maximum(
        jnp.dot(h, k2_ref[...], preferred_element_type=jnp.float32), 0.0)
    out_ref[...] = (
        jnp.dot(xf0_ref[...], kfa_ref[...], preferred_element_type=jnp.float32)
        + jnp.dot(xm0, kfb_ref[...], preferred_element_type=jnp.float32)
        + jnp.dot(m1, kfc_ref[...], preferred_element_type=jnp.float32))


def _final_call(featp, msg0p, aggp, degp, k1t, k1b, k2, kfa, kfb, kfc):
    return pl.pallas_call(
        _final_body,
        grid=(NPK // BR,),
        in_specs=[
            pl.BlockSpec((BR, 128), lambda i: (i, 0)),
            pl.BlockSpec((BR, 128), lambda i: (i, 0)),
            pl.BlockSpec((1, BR, 128), lambda i: (0, i, 0)),
            pl.BlockSpec((1, BR, 128), lambda i: (1, i, 0)),
            pl.BlockSpec((1, BR, 128), lambda i: (0, i, 0)),
            pl.BlockSpec((1, BR, 128), lambda i: (1, i, 0)),
            pl.BlockSpec((128, 256), lambda i: (0, 0)),
            pl.BlockSpec((128, 256), lambda i: (0, 0)),
            pl.BlockSpec((256, 128), lambda i: (0, 0)),
            pl.BlockSpec((128, 256), lambda i: (0, 0)),
            pl.BlockSpec((128, 256), lambda i: (0, 0)),
            pl.BlockSpec((128, 256), lambda i: (0, 0)),
        ],
        out_specs=pl.BlockSpec((BR, 256), lambda i: (i, 0)),
        out_shape=jax.ShapeDtypeStruct((NPK, 256), jnp.float32),
    )(featp, msg0p, aggp, aggp, degp, degp, k1t, k1b, k2, kfa, kfb, kfc)


def _finish_body(pA_ref, pB_ref, cA_ref, cB_ref, fcb_ref, out_ref):
    pool = pA_ref[...] + pB_ref[...]
    cnt = jnp.maximum(cA_ref[...][:, 0] + cB_ref[...][:, 0], 1.0)
    out_ref[...] = pool[:NMOL] / cnt[:NMOL, None] + fcb_ref[...][None, :]


def _finish_call(pA, pB, cA, cB, fc_b):
    return pl.pallas_call(
        _finish_body,
        grid=(1,),
        in_specs=[
            pl.BlockSpec((MOLP, OUT), lambda i: (0, 0)),
            pl.BlockSpec((MOLP, OUT), lambda i: (0, 0)),
            pl.BlockSpec((MOLP, 16), lambda i: (0, 0)),
            pl.BlockSpec((MOLP, 16), lambda i: (0, 0)),
            pl.BlockSpec((OUT,), lambda i: (0,)),
        ],
        out_specs=pl.BlockSpec((NMOL, OUT), lambda i: (0, 0)),
        out_shape=jax.ShapeDtypeStruct((NMOL, OUT), jnp.float32),
    )(pA, pB, cA, cB, fc_b)


def kernel(x, edges_tensor, molecule_ids, embed_table, W_0_0, W_0_1, W_1_0,
           W_1_1, fc_w, fc_b):
    x2d = jnp.pad(x.astype(jnp.int32), (0, N_PAD - N)).reshape(N_PAD // CH, CH)
    mol2d = jnp.pad(molecule_ids.astype(jnp.int32), (0, N_PAD - N),
                    constant_values=NMOL).reshape(N_PAD // CH, CH)
    src2d = jnp.pad(edges_tensor[0].astype(jnp.int32), (0, E_PAD - E),
                    constant_values=N).reshape(E_PAD // CH, CH)
    dst2d = jnp.pad(edges_tensor[1].astype(jnp.int32), (0, E_PAD - E),
                    constant_values=N).reshape(E_PAD // CH, CH)

    i8 = jnp.eye(8, dtype=jnp.float32)
    k0t = jnp.kron(i8, W_0_0[:16])
    k0b = jnp.kron(i8, W_0_0[16:])
    k1 = jnp.kron(i8, W_0_1)
    k1t = jnp.kron(i8, W_1_0[:16])
    k1b = jnp.kron(i8, W_1_0[16:])
    k2 = jnp.kron(i8, W_1_1)
    kfa = jnp.kron(i8, fc_w[0:16])
    kfb = jnp.kron(i8, fc_w[16:32])
    kfc = jnp.kron(i8, fc_w[32:48])

    deg2, cnt2, feat0 = _hist_call(dst2d, mol2d, x2d, embed_table)
    degp = deg2.reshape(NC, NPK, 128)
    featp = feat0.reshape(NPK, 128)

    agg0 = _edge_call(src2d, dst2d, feat0)
    msg0p = _mlp_call(featp, agg0.reshape(NC, NPK, 128), degp, k0t, k0b, k1)
    agg1 = _edge_call(src2d, dst2d, msg0p.reshape(N_PAD, 16))
    yp = _final_call(featp, msg0p, agg1.reshape(NC, NPK, 128), degp, k1t, k1b,
                     k2, kfa, kfb, kfc)
    pool2 = _pool_call(yp.reshape(N_PAD, OUT), mol2d)
    return _finish_call(pool2[0], pool2[1], cnt2[0], cnt2[1], fc_b)
